# Initial kernel scaffold; baseline (speedup 1.0000x reference)
import jax
import jax.numpy as jnp
from jax.experimental import pallas as pl


def kernel(x, edge_index, edge_attr, batch, node_W, node_b, edge_W, edge_b, emlp_W1, emlp_b1, emlp_W2, emlp_b2, struct_scale, conv_eps, conv_W1, conv_b1, conv_W2, conv_b2, gn_alpha, gn_gamma, gn_beta, mean_bio, head_W, head_b):
    raise NotImplementedError("write your pallas kernel here")


# speedup vs baseline: 3.4932x; 3.4932x over previous
import functools

import jax
import jax.numpy as jnp
from jax import lax
from jax.experimental import pallas as pl
from jax.experimental.pallas import tpu as pltpu
from jax.experimental.pallas import tpu_sc as plsc

N = 10000
E = 320000
D = 128
DE = 16
L = 3
G = 64
BIO = 256
HOUT = 64

NC = 2
NS = 16
NW = NC * NS
EPW = E // NW
CB = 80
NCHUNK = EPW // CB
NPAD = 10112
RPT = NPAD // NS


def _msg_body(h_hbm, e_hbm, src_hbm, dst_hbm, zero_hbm, out_hbm,
              sidx, didx, hrows, erows, accum, sem, esem):
    c = lax.axis_index("c")
    s = lax.axis_index("s")
    wid = c * NS + s

    zoff = pl.multiple_of(s * RPT, 8)
    pltpu.sync_copy(zero_hbm.at[pl.ds(zoff, RPT)],
                    accum.at[pl.ds(zoff, RPT)])
    plsc.subcore_barrier()

    def chunk_body(k, _):
        off = pl.multiple_of(wid * EPW + k * CB, 8)
        pltpu.sync_copy(src_hbm.at[pl.ds(off, CB)], sidx)
        pltpu.sync_copy(dst_hbm.at[pl.ds(off, CB)], didx)
        ecp = pltpu.async_copy(e_hbm.at[pl.ds(off, CB)], erows, esem)
        pltpu.async_copy(h_hbm.at[sidx], hrows, sem).wait()
        ecp.wait()

        def row_body(r, _):
            for j in range(D // 16):
                sl = pl.ds(j * 16, 16)
                hrows[r, sl] = jnp.maximum(hrows[r, sl] + erows[r, sl], 0.0)
            return 0

        lax.fori_loop(0, CB, row_body, 0)
        pltpu.sync_copy(hrows, accum.at[didx], add=True)
        return 0

    lax.fori_loop(0, NCHUNK, chunk_body, 0)
    plsc.subcore_barrier()

    pltpu.sync_copy(accum.at[pl.ds(zoff, RPT)],
                    out_hbm.at[c, pl.ds(zoff, RPT)])


@functools.lru_cache(maxsize=1)
def _build_msg_kernel():
    return pl.kernel(
        _msg_body,
        out_type=jax.ShapeDtypeStruct((NC, NPAD, D), jnp.float32),
        mesh=plsc.VectorSubcoreMesh(core_axis_name="c", subcore_axis_name="s",
                                    num_cores=NC, num_subcores=NS),
        scratch_types=[
            pltpu.VMEM((CB,), jnp.int32),
            pltpu.VMEM((CB,), jnp.int32),
            pltpu.VMEM((CB, D), jnp.float32),
            pltpu.VMEM((CB, D), jnp.float32),
            pltpu.VMEM_SHARED((NPAD, D), jnp.float32),
            pltpu.SemaphoreType.DMA,
            pltpu.SemaphoreType.DMA,
        ],
    )


def _node_emb_body(x_ref, w_ref, b_ref, o_ref):
    o_ref[...] = (jnp.dot(x_ref[...], w_ref[...],
                          preferred_element_type=jnp.float32) + b_ref[...])


def _edge_mlp_body(ea_ref, ew_ref, eb_ref, w1_ref, b1_ref, w2_ref, b2_ref,
                   ss_ref, o_ref):
    ea = ea_ref[...]
    e = jnp.dot(ea, ew_ref[...], preferred_element_type=jnp.float32) + eb_ref[...]
    e = jnp.maximum(jnp.dot(e, w1_ref[...],
                            preferred_element_type=jnp.float32) + b1_ref[...], 0.0)
    e = jnp.dot(e, w2_ref[...], preferred_element_type=jnp.float32) + b2_ref[...]
    mask = ea[:, 1:2] > 0.0
    o_ref[...] = jnp.where(mask, e * ss_ref[...], e)


def _layer_body(h_ref, agg_ref, bcol_ref, brow_ref, eps_ref, w1_ref, b1_ref,
                w2_ref, b2_ref, al_ref, ga_ref, be_ref, o_ref):
    h = h_ref[...]
    z = eps_ref[...] * h + agg_ref[0, :N, :] + agg_ref[1, :N, :]
    y = jnp.maximum(jnp.dot(z, w1_ref[...],
                            preferred_element_type=jnp.float32) + b1_ref[...], 0.0)
    y = jnp.dot(y, w2_ref[...], preferred_element_type=jnp.float32) + b2_ref[...]

    oh = (bcol_ref[...] == lax.broadcasted_iota(jnp.int32, (N, G), 1)
          ).astype(jnp.float32)
    oht = (brow_ref[...] == lax.broadcasted_iota(jnp.int32, (G, N), 0)
           ).astype(jnp.float32)
    inv_cnt = 1.0 / jnp.maximum(jnp.sum(oht, axis=1, keepdims=True), 1.0)

    mean = jnp.dot(oht, y, preferred_element_type=jnp.float32) * inv_cnt
    hc = y - al_ref[...] * jnp.dot(oh, mean, preferred_element_type=jnp.float32)
    var = jnp.dot(oht, hc * hc, preferred_element_type=jnp.float32) * inv_cnt
    vb = jnp.dot(oh, var, preferred_element_type=jnp.float32)
    o_ref[...] = ga_ref[...] * hc * lax.rsqrt(vb + 1e-5) + be_ref[...]


def _head_body(h_ref, brow_ref, bio_ref, hw_ref, hb_ref, o_ref):
    oht = (brow_ref[...] == lax.broadcasted_iota(jnp.int32, (G, N), 0)
           ).astype(jnp.float32)
    inv_cnt = 1.0 / jnp.maximum(jnp.sum(oht, axis=1, keepdims=True), 1.0)
    g = jnp.dot(oht, h_ref[...], preferred_element_type=jnp.float32) * inv_cnt
    combined = jnp.concatenate(
        [g, jnp.broadcast_to(bio_ref[...], (G, BIO))], axis=1)
    out = jnp.dot(combined, hw_ref[...],
                  preferred_element_type=jnp.float32) + hb_ref[...]
    o_ref[...] = jnp.mean(out, axis=1, keepdims=True)


BE = 3200


def _edge_mlp(edge_attr, edge_W, edge_b, W1, b1, W2, b2, ssrow):
    grid = (E // BE,)
    full = lambda shape: pl.BlockSpec(shape, lambda i: (0, 0))
    return pl.pallas_call(
        _edge_mlp_body,
        grid=grid,
        in_specs=[
            pl.BlockSpec((BE, DE), lambda i: (i, 0)),
            full((DE, D)), full((1, D)), full((D, D)), full((1, D)),
            full((D, D)), full((1, D)), full((1, D)),
        ],
        out_specs=pl.BlockSpec((BE, D), lambda i: (i, 0)),
        out_shape=jax.ShapeDtypeStruct((E, D), jnp.float32),
    )(edge_attr, edge_W, edge_b, W1, b1, W2, b2, ssrow)


def kernel(x, edge_index, edge_attr, batch, node_W, node_b, edge_W, edge_b,
           emlp_W1, emlp_b1, emlp_W2, emlp_b2, struct_scale, conv_eps,
           conv_W1, conv_b1, conv_W2, conv_b2, gn_alpha, gn_gamma, gn_beta,
           mean_bio, head_W, head_b):
    f32 = jnp.float32
    row = lambda v: v.reshape(1, -1).astype(f32)
    src = edge_index[0].astype(jnp.int32)
    dst = edge_index[1].astype(jnp.int32)
    bcol = batch.astype(jnp.int32).reshape(N, 1)
    brow = batch.astype(jnp.int32).reshape(1, N)
    zeros_nd = jnp.zeros((NPAD, D), f32)
    ssrow = jnp.broadcast_to(struct_scale.astype(f32).reshape(1, 1), (1, D))

    h = pl.pallas_call(
        _node_emb_body,
        out_shape=jax.ShapeDtypeStruct((N, D), f32),
    )(x, node_W, row(node_b))

    e = _edge_mlp(edge_attr, edge_W, row(edge_b), emlp_W1, row(emlp_b1),
                  emlp_W2, row(emlp_b2), ssrow)

    for l in range(L):
        agg = _build_msg_kernel()(h, e, src, dst, zeros_nd)
        epsrow = jnp.broadcast_to((1.0 + conv_eps[l]).reshape(1, 1), (1, D))
        h = pl.pallas_call(
            _layer_body,
            out_shape=jax.ShapeDtypeStruct((N, D), f32),
        )(h, agg, bcol, brow, epsrow, conv_W1[l], row(conv_b1[l]),
          conv_W2[l], row(conv_b2[l]), row(gn_alpha[l]), row(gn_gamma[l]),
          row(gn_beta[l]))

    out = pl.pallas_call(
        _head_body,
        out_shape=jax.ShapeDtypeStruct((G, 1), f32),
    )(h, brow, row(mean_bio), head_W, row(head_b))
    return out


# speedup vs baseline: 5.7452x; 1.6447x over previous
import functools

import jax
import jax.numpy as jnp
from jax import lax
from jax.experimental import pallas as pl
from jax.experimental.pallas import tpu as pltpu
from jax.experimental.pallas import tpu_sc as plsc

N = 10000
E = 320000
D = 128
DE = 16
L = 3
G = 64
BIO = 256
HOUT = 64

NC = 2
NS = 16
NW = NC * NS
EPW = E // NW
CB = 80
NCHUNK = EPW // CB
NPAD = 10112
RPT = NPAD // NS


def _msg_body(h_hbm, e_hbm, src_hbm, dst_hbm, zero_hbm, out_hbm,
              sidx, didx, hrows, erows, gsem, esem, isem, zsem, accum):
    c = lax.axis_index("c")
    s = lax.axis_index("s")
    wid = c * NS + s
    base = wid * EPW

    zoff = pl.multiple_of(s * RPT, 8)
    zcp = pltpu.async_copy(zero_hbm.at[pl.ds(zoff, RPT)],
                           accum.at[pl.ds(zoff, RPT)], zsem)

    def start_idx(k, b):
        off = pl.multiple_of(base + k * CB, 8)
        pltpu.async_copy(src_hbm.at[pl.ds(off, CB)], sidx.at[b], isem.at[b])
        pltpu.async_copy(dst_hbm.at[pl.ds(off, CB)], didx.at[b], isem.at[b])

    def wait_idx(b):
        pltpu.make_async_copy(src_hbm.at[pl.ds(0, CB)], sidx.at[b], isem.at[b]).wait()
        pltpu.make_async_copy(dst_hbm.at[pl.ds(0, CB)], didx.at[b], isem.at[b]).wait()

    def start_rows(k, b):
        off = pl.multiple_of(base + k * CB, 8)
        pltpu.async_copy(e_hbm.at[pl.ds(off, CB)], erows.at[b], esem.at[b])
        pltpu.async_copy(h_hbm.at[sidx.at[b]], hrows.at[b], gsem.at[b])

    def wait_rows(b):
        pltpu.make_async_copy(e_hbm.at[pl.ds(0, CB)], erows.at[b], esem.at[b]).wait()
        pltpu.make_async_copy(h_hbm.at[pl.ds(0, CB)], hrows.at[b], gsem.at[b]).wait()

    def compute_scatter(b):
        def row_body(r, _):
            for j in range(D // 16):
                sl = pl.ds(j * 16, 16)
                hrows[b, r, sl] = jnp.maximum(
                    hrows[b, r, sl] + erows[b, r, sl], 0.0)
            return 0

        lax.fori_loop(0, CB, row_body, 0)
        pltpu.sync_copy(hrows.at[b], accum.at[didx.at[b]], add=True)

    start_idx(0, 0)
    wait_idx(0)
    zcp.wait()
    plsc.subcore_barrier()
    start_rows(0, 0)

    def pipe_body(g, _):
        for b in range(2):
            k = g * 2 + b
            start_idx(k + 1, 1 - b)
            wait_rows(b)
            wait_idx(1 - b)
            start_rows(k + 1, 1 - b)
            compute_scatter(b)
        return 0

    lax.fori_loop(0, (NCHUNK - 1) // 2, pipe_body, 0)

    wait_rows(0)
    compute_scatter(0)
    plsc.subcore_barrier()

    pltpu.sync_copy(accum.at[pl.ds(zoff, RPT)],
                    out_hbm.at[c, pl.ds(zoff, RPT)])


@functools.lru_cache(maxsize=1)
def _build_msg_kernel():
    return pl.kernel(
        _msg_body,
        out_type=jax.ShapeDtypeStruct((NC, NPAD, D), jnp.float32),
        mesh=plsc.VectorSubcoreMesh(core_axis_name="c", subcore_axis_name="s",
                                    num_cores=NC, num_subcores=NS),
        scratch_types=[
            pltpu.VMEM((2, CB), jnp.int32),
            pltpu.VMEM((2, CB), jnp.int32),
            pltpu.VMEM((2, CB, D), jnp.float32),
            pltpu.VMEM((2, CB, D), jnp.float32),
            pltpu.SemaphoreType.DMA((2,)),
            pltpu.SemaphoreType.DMA((2,)),
            pltpu.SemaphoreType.DMA((2,)),
            pltpu.SemaphoreType.DMA,
            pltpu.VMEM_SHARED((NPAD, D), jnp.float32),
        ],
    )


def _node_emb_body(x_ref, w_ref, b_ref, o_ref):
    o_ref[...] = (jnp.dot(x_ref[...], w_ref[...],
                          preferred_element_type=jnp.float32) + b_ref[...])


def _edge_mlp_body(ea_ref, ew_ref, eb_ref, w1_ref, b1_ref, w2_ref, b2_ref,
                   ss_ref, o_ref):
    ea = ea_ref[...]
    e = jnp.dot(ea, ew_ref[...], preferred_element_type=jnp.float32) + eb_ref[...]
    e = jnp.maximum(jnp.dot(e, w1_ref[...],
                            preferred_element_type=jnp.float32) + b1_ref[...], 0.0)
    e = jnp.dot(e, w2_ref[...], preferred_element_type=jnp.float32) + b2_ref[...]
    mask = ea[:, 1:2] > 0.0
    o_ref[...] = jnp.where(mask, e * ss_ref[...], e)


def _layer_body(h_ref, agg_ref, bcol_ref, brow_ref, eps_ref, w1_ref, b1_ref,
                w2_ref, b2_ref, al_ref, ga_ref, be_ref, o_ref):
    h = h_ref[...]
    z = eps_ref[...] * h + agg_ref[0, :N, :] + agg_ref[1, :N, :]
    y = jnp.maximum(jnp.dot(z, w1_ref[...],
                            preferred_element_type=jnp.float32) + b1_ref[...], 0.0)
    y = jnp.dot(y, w2_ref[...], preferred_element_type=jnp.float32) + b2_ref[...]

    oh = (bcol_ref[...] == lax.broadcasted_iota(jnp.int32, (N, G), 1)
          ).astype(jnp.float32)
    oht = (brow_ref[...] == lax.broadcasted_iota(jnp.int32, (G, N), 0)
           ).astype(jnp.float32)
    inv_cnt = 1.0 / jnp.maximum(jnp.sum(oht, axis=1, keepdims=True), 1.0)

    mean = jnp.dot(oht, y, preferred_element_type=jnp.float32) * inv_cnt
    hc = y - al_ref[...] * jnp.dot(oh, mean, preferred_element_type=jnp.float32)
    var = jnp.dot(oht, hc * hc, preferred_element_type=jnp.float32) * inv_cnt
    vb = jnp.dot(oh, var, preferred_element_type=jnp.float32)
    o_ref[...] = ga_ref[...] * hc * lax.rsqrt(vb + 1e-5) + be_ref[...]


def _head_body(h_ref, brow_ref, bio_ref, hw_ref, hb_ref, o_ref):
    oht = (brow_ref[...] == lax.broadcasted_iota(jnp.int32, (G, N), 0)
           ).astype(jnp.float32)
    inv_cnt = 1.0 / jnp.maximum(jnp.sum(oht, axis=1, keepdims=True), 1.0)
    g = jnp.dot(oht, h_ref[...], preferred_element_type=jnp.float32) * inv_cnt
    combined = jnp.concatenate(
        [g, jnp.broadcast_to(bio_ref[...], (G, BIO))], axis=1)
    out = jnp.dot(combined, hw_ref[...],
                  preferred_element_type=jnp.float32) + hb_ref[...]
    o_ref[...] = jnp.mean(out, axis=1, keepdims=True)


BE = 3200


def _edge_mlp(edge_attr, edge_W, edge_b, W1, b1, W2, b2, ssrow):
    grid = (E // BE,)
    full = lambda shape: pl.BlockSpec(shape, lambda i: (0, 0))
    return pl.pallas_call(
        _edge_mlp_body,
        grid=grid,
        in_specs=[
            pl.BlockSpec((BE, DE), lambda i: (i, 0)),
            full((DE, D)), full((1, D)), full((D, D)), full((1, D)),
            full((D, D)), full((1, D)), full((1, D)),
        ],
        out_specs=pl.BlockSpec((BE, D), lambda i: (i, 0)),
        out_shape=jax.ShapeDtypeStruct((E, D), jnp.float32),
    )(edge_attr, edge_W, edge_b, W1, b1, W2, b2, ssrow)


def kernel(x, edge_index, edge_attr, batch, node_W, node_b, edge_W, edge_b,
           emlp_W1, emlp_b1, emlp_W2, emlp_b2, struct_scale, conv_eps,
           conv_W1, conv_b1, conv_W2, conv_b2, gn_alpha, gn_gamma, gn_beta,
           mean_bio, head_W, head_b):
    f32 = jnp.float32
    row = lambda v: v.reshape(1, -1).astype(f32)
    src = edge_index[0].astype(jnp.int32)
    dst = edge_index[1].astype(jnp.int32)
    bcol = batch.astype(jnp.int32).reshape(N, 1)
    brow = batch.astype(jnp.int32).reshape(1, N)
    zeros_nd = jnp.zeros((NPAD, D), f32)
    ssrow = jnp.broadcast_to(struct_scale.astype(f32).reshape(1, 1), (1, D))

    h = pl.pallas_call(
        _node_emb_body,
        out_shape=jax.ShapeDtypeStruct((N, D), f32),
    )(x, node_W, row(node_b))

    e = _edge_mlp(edge_attr, edge_W, row(edge_b), emlp_W1, row(emlp_b1),
                  emlp_W2, row(emlp_b2), ssrow)

    for l in range(L):
        agg = _build_msg_kernel()(h, e, src, dst, zeros_nd)
        epsrow = jnp.broadcast_to((1.0 + conv_eps[l]).reshape(1, 1), (1, D))
        h = pl.pallas_call(
            _layer_body,
            out_shape=jax.ShapeDtypeStruct((N, D), f32),
        )(h, agg, bcol, brow, epsrow, conv_W1[l], row(conv_b1[l]),
          conv_W2[l], row(conv_b2[l]), row(gn_alpha[l]), row(gn_gamma[l]),
          row(gn_beta[l]))

    out = pl.pallas_call(
        _head_body,
        out_shape=jax.ShapeDtypeStruct((G, 1), f32),
    )(h, brow, row(mean_bio), head_W, row(head_b))
    return out


# speedup vs baseline: 5.8459x; 1.0175x over previous
import functools

import jax
import jax.numpy as jnp
from jax import lax
from jax.experimental import pallas as pl
from jax.experimental.pallas import tpu as pltpu
from jax.experimental.pallas import tpu_sc as plsc

N = 10000
E = 320000
D = 128
DE = 16
L = 3
G = 64
BIO = 256
HOUT = 64

NC = 2
NS = 16
NW = NC * NS
EPW = E // NW
CB = 80
NCHUNK = EPW // CB
NPAD = 10112
RPT = NPAD // NS


def _msg_body(h_hbm, e_hbm, ei_hbm, zero_hbm, out_hbm,
              ibuf, hrows, erows, gsem, esem, isem, zsem, accum):
    c = lax.axis_index("c")
    s = lax.axis_index("s")
    wid = c * NS + s
    base = wid * EPW

    zoff = pl.multiple_of(s * RPT, 8)
    zcp = pltpu.async_copy(zero_hbm.at[pl.ds(zoff, RPT)],
                           accum.at[pl.ds(zoff, RPT)], zsem)

    def start_idx(k, b):
        off = pl.multiple_of(base + k * CB, 8)
        pltpu.async_copy(ei_hbm.at[pl.ds(off, CB)], ibuf.at[b, 0], isem.at[b])
        pltpu.async_copy(ei_hbm.at[pl.ds(E + off, CB)], ibuf.at[b, 1],
                         isem.at[b])

    def wait_idx(b):
        pltpu.make_async_copy(ei_hbm.at[pl.ds(0, CB)], ibuf.at[b, 0],
                              isem.at[b]).wait()
        pltpu.make_async_copy(ei_hbm.at[pl.ds(0, CB)], ibuf.at[b, 1],
                              isem.at[b]).wait()

    def start_rows(k, b):
        off = pl.multiple_of(base + k * CB, 8)
        pltpu.async_copy(e_hbm.at[pl.ds(off, CB)], erows.at[b], esem.at[b])
        pltpu.async_copy(h_hbm.at[ibuf.at[b, 0]], hrows.at[b], gsem.at[b])

    def wait_rows(b):
        pltpu.make_async_copy(e_hbm.at[pl.ds(0, CB)], erows.at[b], esem.at[b]).wait()
        pltpu.make_async_copy(h_hbm.at[pl.ds(0, CB)], hrows.at[b], gsem.at[b]).wait()

    def compute_scatter(b):
        def row_body(r, _):
            for j in range(D // 16):
                sl = pl.ds(j * 16, 16)
                hrows[b, r, sl] = jnp.maximum(
                    hrows[b, r, sl] + erows[b, r, sl], 0.0)
            return 0

        lax.fori_loop(0, CB, row_body, 0)
        pltpu.sync_copy(hrows.at[b], accum.at[ibuf.at[b, 1]], add=True)

    start_idx(0, 0)
    wait_idx(0)
    zcp.wait()
    plsc.subcore_barrier()
    start_rows(0, 0)

    def pipe_body(g, _):
        for b in range(2):
            k = g * 2 + b
            start_idx(k + 1, 1 - b)
            wait_rows(b)
            wait_idx(1 - b)
            start_rows(k + 1, 1 - b)
            compute_scatter(b)
        return 0

    lax.fori_loop(0, (NCHUNK - 1) // 2, pipe_body, 0)

    wait_rows(0)
    compute_scatter(0)
    plsc.subcore_barrier()

    pltpu.sync_copy(accum.at[pl.ds(zoff, RPT)],
                    out_hbm.at[c, pl.ds(zoff, RPT)])


@functools.lru_cache(maxsize=1)
def _build_msg_kernel():
    return pl.kernel(
        _msg_body,
        out_type=jax.ShapeDtypeStruct((NC, NPAD, D), jnp.float32),
        mesh=plsc.VectorSubcoreMesh(core_axis_name="c", subcore_axis_name="s",
                                    num_cores=NC, num_subcores=NS),
        scratch_types=[
            pltpu.VMEM((2, 2, CB), jnp.int32),
            pltpu.VMEM((2, CB, D), jnp.float32),
            pltpu.VMEM((2, CB, D), jnp.float32),
            pltpu.SemaphoreType.DMA((2,)),
            pltpu.SemaphoreType.DMA((2,)),
            pltpu.SemaphoreType.DMA((2,)),
            pltpu.SemaphoreType.DMA,
            pltpu.VMEM_SHARED((NPAD, D), jnp.float32),
        ],
    )


def _node_emb_body(x_ref, w_ref, b_ref, o_ref):
    o_ref[...] = (jnp.dot(x_ref[...], w_ref[...],
                          preferred_element_type=jnp.float32) + b_ref[...])


def _edge_mlp_body(ea_ref, ew_ref, eb_ref, w1_ref, b1_ref, w2_ref, b2_ref,
                   ss_ref, o_ref):
    ea = ea_ref[...]
    w0 = jnp.dot(ew_ref[...], w1_ref[...], preferred_element_type=jnp.float32)
    b0 = jnp.dot(eb_ref[...], w1_ref[...],
                 preferred_element_type=jnp.float32) + b1_ref[...]
    e = jnp.maximum(jnp.dot(ea, w0, preferred_element_type=jnp.float32) + b0, 0.0)
    e = jnp.dot(e, w2_ref[...], preferred_element_type=jnp.float32) + b2_ref[...]
    mask = ea[:, 1:2] > 0.0
    o_ref[...] = jnp.where(mask, e * ss_ref[...], e)


def _layer_body(h_ref, agg_ref, bcol_ref, brow_ref, eps_ref, w1_ref, b1_ref,
                w2_ref, b2_ref, al_ref, ga_ref, be_ref, o_ref):
    h = h_ref[...]
    z = eps_ref[...] * h + agg_ref[0, :N, :] + agg_ref[1, :N, :]
    y = jnp.maximum(jnp.dot(z, w1_ref[...],
                            preferred_element_type=jnp.float32) + b1_ref[...], 0.0)
    y = jnp.dot(y, w2_ref[...], preferred_element_type=jnp.float32) + b2_ref[...]

    oh = (bcol_ref[...] == lax.broadcasted_iota(jnp.int32, (N, G), 1)
          ).astype(jnp.float32)
    oht = (brow_ref[...] == lax.broadcasted_iota(jnp.int32, (G, N), 0)
           ).astype(jnp.float32)
    inv_cnt = 1.0 / jnp.maximum(jnp.sum(oht, axis=1, keepdims=True), 1.0)

    mean = jnp.dot(oht, y, preferred_element_type=jnp.float32) * inv_cnt
    hc = y - al_ref[...] * jnp.dot(oh, mean, preferred_element_type=jnp.float32)
    var = jnp.dot(oht, hc * hc, preferred_element_type=jnp.float32) * inv_cnt
    vb = jnp.dot(oh, var, preferred_element_type=jnp.float32)
    o_ref[...] = ga_ref[...] * hc * lax.rsqrt(vb + 1e-5) + be_ref[...]


def _head_body(h_ref, brow_ref, bio_ref, hw_ref, hb_ref, o_ref):
    oht = (brow_ref[...] == lax.broadcasted_iota(jnp.int32, (G, N), 0)
           ).astype(jnp.float32)
    inv_cnt = 1.0 / jnp.maximum(jnp.sum(oht, axis=1, keepdims=True), 1.0)
    g = jnp.dot(oht, h_ref[...], preferred_element_type=jnp.float32) * inv_cnt
    combined = jnp.concatenate(
        [g, jnp.broadcast_to(bio_ref[...], (G, BIO))], axis=1)
    out = jnp.dot(combined, hw_ref[...],
                  preferred_element_type=jnp.float32) + hb_ref[...]
    o_ref[...] = jnp.mean(out, axis=1, keepdims=True)


BE = 3200


def _edge_mlp(edge_attr, edge_W, edge_b, W1, b1, W2, b2, ssrow):
    grid = (E // BE,)
    full = lambda shape: pl.BlockSpec(shape, lambda i: (0, 0))
    return pl.pallas_call(
        _edge_mlp_body,
        grid=grid,
        in_specs=[
            pl.BlockSpec((BE, DE), lambda i: (i, 0)),
            full((DE, D)), full((1, D)), full((D, D)), full((1, D)),
            full((D, D)), full((1, D)), full((1, D)),
        ],
        out_specs=pl.BlockSpec((BE, D), lambda i: (i, 0)),
        out_shape=jax.ShapeDtypeStruct((E, D), jnp.float32),
    )(edge_attr, edge_W, edge_b, W1, b1, W2, b2, ssrow)


def kernel(x, edge_index, edge_attr, batch, node_W, node_b, edge_W, edge_b,
           emlp_W1, emlp_b1, emlp_W2, emlp_b2, struct_scale, conv_eps,
           conv_W1, conv_b1, conv_W2, conv_b2, gn_alpha, gn_gamma, gn_beta,
           mean_bio, head_W, head_b):
    f32 = jnp.float32
    row = lambda v: v.reshape(1, -1).astype(f32)
    ei = edge_index.astype(jnp.int32).reshape(-1)
    bcol = batch.astype(jnp.int32).reshape(N, 1)
    brow = batch.astype(jnp.int32).reshape(1, N)
    zeros_nd = jnp.zeros((NPAD, D), f32)
    ssrow = jnp.broadcast_to(struct_scale.astype(f32).reshape(1, 1), (1, D))

    h = pl.pallas_call(
        _node_emb_body,
        out_shape=jax.ShapeDtypeStruct((N, D), f32),
    )(x, node_W, row(node_b))

    e = _edge_mlp(edge_attr, edge_W, row(edge_b), emlp_W1, row(emlp_b1),
                  emlp_W2, row(emlp_b2), ssrow)

    for l in range(L):
        agg = _build_msg_kernel()(h, e, ei, zeros_nd)
        epsrow = jnp.broadcast_to((1.0 + conv_eps[l]).reshape(1, 1), (1, D))
        h = pl.pallas_call(
            _layer_body,
            out_shape=jax.ShapeDtypeStruct((N, D), f32),
        )(h, agg, bcol, brow, epsrow, conv_W1[l], row(conv_b1[l]),
          conv_W2[l], row(conv_b2[l]), row(gn_alpha[l]), row(gn_gamma[l]),
          row(gn_beta[l]))

    out = pl.pallas_call(
        _head_body,
        out_shape=jax.ShapeDtypeStruct((G, 1), f32),
    )(h, brow, row(mean_bio), head_W, row(head_b))
    return out
